# Initial kernel scaffold; baseline (speedup 1.0000x reference)
import jax
import jax.numpy as jnp
from jax.experimental import pallas as pl


def kernel(w1, b1, w2, b2, wproj, bproj, x):
    raise NotImplementedError("write your pallas kernel here")


# speedup vs baseline: 11.7820x; 11.7820x over previous
import math

import jax
import jax.numpy as jnp
from jax.experimental import pallas as pl
from jax.experimental.pallas import tpu as pltpu

INV_SQRT2 = 1.0 / math.sqrt(2.0)
NEG_SLOPE = 0.2
HIGHEST = jax.lax.Precision.HIGHEST

H = 32
W = 32
HW = H * W
PLANES = 32
TAPS = 9


def _lrelu(v):
    return jnp.where(v >= 0.0, v, NEG_SLOPE * v)


def _combine_taps(parts):
    q = jax.lax.broadcasted_iota(jnp.int32, (PLANES, HW), 1)
    hh = q // W
    ww = q % W
    acc = None
    for t in range(TAPS):
        dy = t // 3 - 1
        dx = t % 3 - 1
        z = parts[t * PLANES:(t + 1) * PLANES, :]
        off = dy * W + dx
        if off != 0:
            z = jnp.roll(z, -off, axis=1)
        cond = None
        for c in ((hh >= 1) if dy == -1 else None,
                  (hh <= H - 2) if dy == 1 else None,
                  (ww >= 1) if dx == -1 else None,
                  (ww <= W - 2) if dx == 1 else None):
            if c is not None:
                cond = c if cond is None else (cond & c)
        if cond is not None:
            z = jnp.where(cond, z, 0.0)
        acc = z if acc is None else acc + z
    return acc


def _block_kernel(x_ref, wbig_ref, wc1_ref, w2_ref, wc2_ref, bias_ref,
                  coords_ref, o_ref):
    x = x_ref[0]
    coords = coords_ref[...]

    a = jnp.dot(wbig_ref[...], x,
                preferred_element_type=jnp.float32, precision=HIGHEST)
    a = a + jnp.dot(wc1_ref[...], coords,
                    preferred_element_type=jnp.float32, precision=HIGHEST)

    b1 = bias_ref[:, 0:1]
    b2 = bias_ref[:, 1:2]
    bp = bias_ref[:, 2:3]

    y1 = _lrelu(_combine_taps(a[:TAPS * PLANES]) + b1)

    b = jnp.dot(w2_ref[...], y1,
                preferred_element_type=jnp.float32, precision=HIGHEST)
    b = b + jnp.dot(wc2_ref[...], coords,
                    preferred_element_type=jnp.float32, precision=HIGHEST)
    y2 = _lrelu(_combine_taps(b) + b2)

    ident = a[TAPS * PLANES:TAPS * PLANES + PLANES] + bp
    o_ref[0] = (y2 + ident) * INV_SQRT2


def _tap_major(w):
    cout, cin = w.shape[0], w.shape[1]
    return w.transpose(2, 3, 0, 1).reshape(TAPS * cout, cin)


def kernel(w1, b1, w2, b2, wproj, bproj, x):
    B, Cin = x.shape[0], x.shape[1]
    x3 = x.astype(jnp.float32).reshape(B, Cin, HW)

    w1f = w1.astype(jnp.float32)
    w2f = w2.astype(jnp.float32)

    w1_main = _tap_major(w1f[:, :Cin])
    w1_coord = _tap_major(w1f[:, Cin:])
    wbig = jnp.concatenate(
        [w1_main, wproj.astype(jnp.float32).reshape(PLANES, Cin)], axis=0)
    wc1 = jnp.pad(w1_coord, ((0, PLANES), (0, 6)))

    w2_main = _tap_major(w2f[:, :PLANES])
    wc2 = jnp.pad(_tap_major(w2f[:, PLANES:]), ((0, 0), (0, 6)))

    bias = jnp.stack([b1, b2, bproj], axis=1).astype(jnp.float32)
    bias = jnp.pad(bias, ((0, 0), (0, 5)))

    span = jnp.arange(H, dtype=jnp.float32) / (H - 1) * 2.0 - 1.0
    xx = jnp.broadcast_to(span[:, None], (H, W)).reshape(1, HW)
    yy = jnp.broadcast_to(span[None, :], (H, W)).reshape(1, HW)
    coords = jnp.concatenate([xx, yy, jnp.zeros((6, HW), jnp.float32)], axis=0)

    out = pl.pallas_call(
        _block_kernel,
        grid=(B,),
        out_shape=jax.ShapeDtypeStruct((B, PLANES, HW), jnp.float32),
        in_specs=[
            pl.BlockSpec((1, Cin, HW), lambda i: (i, 0, 0)),
            pl.BlockSpec(wbig.shape, lambda i: (0, 0)),
            pl.BlockSpec(wc1.shape, lambda i: (0, 0)),
            pl.BlockSpec(w2_main.shape, lambda i: (0, 0)),
            pl.BlockSpec(wc2.shape, lambda i: (0, 0)),
            pl.BlockSpec(bias.shape, lambda i: (0, 0)),
            pl.BlockSpec(coords.shape, lambda i: (0, 0)),
        ],
        out_specs=pl.BlockSpec((1, PLANES, HW), lambda i: (i, 0, 0)),
        compiler_params=pltpu.CompilerParams(
            dimension_semantics=("parallel",)),
    )(x3, wbig, wc1, w2_main, wc2, bias, coords)

    return out.reshape(B, PLANES, H, W)


# speedup vs baseline: 21.9271x; 1.8611x over previous
import math

import jax
import jax.numpy as jnp
from jax.experimental import pallas as pl
from jax.experimental.pallas import tpu as pltpu

INV_SQRT2 = 1.0 / math.sqrt(2.0)
NEG_SLOPE = 0.2
MM_PREC = jax.lax.Precision.DEFAULT

H = 32
W = 32
HW = H * W
PLANES = 32
TAPS = 9


def _lrelu(v):
    return jnp.where(v >= 0.0, v, NEG_SLOPE * v)


def _combine_taps(parts):
    q = jax.lax.broadcasted_iota(jnp.int32, (PLANES, HW), 1)
    hh = q // W
    ww = q % W
    acc = None
    for t in range(TAPS):
        dy = t // 3 - 1
        dx = t % 3 - 1
        z = parts[t * PLANES:(t + 1) * PLANES, :]
        off = dy * W + dx
        if off != 0:
            z = jnp.roll(z, -off, axis=1)
        cond = None
        for c in ((hh >= 1) if dy == -1 else None,
                  (hh <= H - 2) if dy == 1 else None,
                  (ww >= 1) if dx == -1 else None,
                  (ww <= W - 2) if dx == 1 else None):
            if c is not None:
                cond = c if cond is None else (cond & c)
        if cond is not None:
            z = jnp.where(cond, z, 0.0)
        acc = z if acc is None else acc + z
    return acc


def _block_kernel(x_ref, wbig_ref, wc1_ref, w2_ref, wc2_ref, bias_ref,
                  coords_ref, o_ref):
    x = x_ref[0]
    coords = coords_ref[...]

    a = jnp.dot(wbig_ref[...], x,
                preferred_element_type=jnp.float32, precision=MM_PREC)
    a = a + jnp.dot(wc1_ref[...], coords,
                    preferred_element_type=jnp.float32, precision=MM_PREC)

    b1 = bias_ref[:, 0:1]
    b2 = bias_ref[:, 1:2]
    bp = bias_ref[:, 2:3]

    y1 = _lrelu(_combine_taps(a[:TAPS * PLANES]) + b1)

    b = jnp.dot(w2_ref[...], y1,
                preferred_element_type=jnp.float32, precision=MM_PREC)
    b = b + jnp.dot(wc2_ref[...], coords,
                    preferred_element_type=jnp.float32, precision=MM_PREC)
    y2 = _lrelu(_combine_taps(b) + b2)

    ident = a[TAPS * PLANES:TAPS * PLANES + PLANES] + bp
    o_ref[0] = (y2 + ident) * INV_SQRT2


def _tap_major(w):
    cout, cin = w.shape[0], w.shape[1]
    return w.transpose(2, 3, 0, 1).reshape(TAPS * cout, cin)


def kernel(w1, b1, w2, b2, wproj, bproj, x):
    B, Cin = x.shape[0], x.shape[1]
    x3 = x.astype(jnp.float32).reshape(B, Cin, HW)

    w1f = w1.astype(jnp.float32)
    w2f = w2.astype(jnp.float32)

    w1_main = _tap_major(w1f[:, :Cin])
    w1_coord = _tap_major(w1f[:, Cin:])
    wbig = jnp.concatenate(
        [w1_main, wproj.astype(jnp.float32).reshape(PLANES, Cin)], axis=0)
    wc1 = jnp.pad(w1_coord, ((0, PLANES), (0, 6)))

    w2_main = _tap_major(w2f[:, :PLANES])
    wc2 = jnp.pad(_tap_major(w2f[:, PLANES:]), ((0, 0), (0, 6)))

    bias = jnp.stack([b1, b2, bproj], axis=1).astype(jnp.float32)
    bias = jnp.pad(bias, ((0, 0), (0, 5)))

    span = jnp.arange(H, dtype=jnp.float32) / (H - 1) * 2.0 - 1.0
    xx = jnp.broadcast_to(span[:, None], (H, W)).reshape(1, HW)
    yy = jnp.broadcast_to(span[None, :], (H, W)).reshape(1, HW)
    coords = jnp.concatenate([xx, yy, jnp.zeros((6, HW), jnp.float32)], axis=0)

    out = pl.pallas_call(
        _block_kernel,
        grid=(B,),
        out_shape=jax.ShapeDtypeStruct((B, PLANES, HW), jnp.float32),
        in_specs=[
            pl.BlockSpec((1, Cin, HW), lambda i: (i, 0, 0)),
            pl.BlockSpec(wbig.shape, lambda i: (0, 0)),
            pl.BlockSpec(wc1.shape, lambda i: (0, 0)),
            pl.BlockSpec(w2_main.shape, lambda i: (0, 0)),
            pl.BlockSpec(wc2.shape, lambda i: (0, 0)),
            pl.BlockSpec(bias.shape, lambda i: (0, 0)),
            pl.BlockSpec(coords.shape, lambda i: (0, 0)),
        ],
        out_specs=pl.BlockSpec((1, PLANES, HW), lambda i: (i, 0, 0)),
        compiler_params=pltpu.CompilerParams(
            dimension_semantics=("parallel",)),
    )(x3, wbig, wc1, w2_main, wc2, bias, coords)

    return out.reshape(B, PLANES, H, W)
